# baseline (speedup 1.0000x reference)
import functools

import jax
import jax.numpy as jnp
from jax import lax
from jax.experimental import pallas as pl
from jax.experimental.pallas import tpu as pltpu
from jax.experimental.pallas import tpu_sc as plsc

N = 10000
E = 320000
IN_DIM = 128
HID_DIM = 128
OUT_DIM = 64
HALF = IN_DIM // 2

NPAD = 10240
EPAD = 327680
K = 128
BLOCKS_PER_TILE = 160
EROWS = EPAD // K
ROWS_PER_TILE = NPAD // 16
TC_BLK = 1280


def _sc_aggregate(x_cols, src_both, dst2d):
    mesh = plsc.VectorSubcoreMesh(core_axis_name="c", subcore_axis_name="s")

    @functools.partial(
        pl.kernel,
        mesh=mesh,
        compiler_params=pltpu.CompilerParams(use_tc_tiling_on_sc=False),
        out_type=[
            jax.ShapeDtypeStruct((2 * NPAD, HALF), jnp.float32),
            jax.ShapeDtypeStruct((2 * NPAD,), jnp.float32),
        ],
        scratch_types=[
            pltpu.VMEM((BLOCKS_PER_TILE, K), jnp.int32),
            pltpu.VMEM((BLOCKS_PER_TILE, K), jnp.int32),
            pltpu.VMEM((K, HALF), jnp.float32),
            pltpu.VMEM((K, HALF), jnp.float32),
            pltpu.VMEM((K, HALF), jnp.float32),
            pltpu.VMEM((K, HALF), jnp.float32),
            pltpu.VMEM((K,), jnp.float32),
            pltpu.VMEM((K, HALF), jnp.float32),
            pltpu.VMEM((2 * ROWS_PER_TILE,), jnp.float32),
            pltpu.VMEM_SHARED((NPAD, HALF), jnp.float32),
            pltpu.VMEM_SHARED((2 * NPAD,), jnp.float32),
        ] + [pltpu.SemaphoreType.DMA] * 9,
    )
    def agg(x_hbm, src_hbm, dst_hbm, xagg_out, c_out,
            gidx, sidx, rows0, rows1, rows2, rows3, ones, zbuf, zvec,
            xagg_sh, hist_sh, sem_g0, sem_g1, sem_g2, sem_g3,
            sem_s0, sem_s1, sem_s2, sem_s3, sem_h):
        cid = lax.axis_index("c")
        sid = lax.axis_index("s")

        def fill(i, _):
            r = i // (HALF // 16)
            col = (i % (HALF // 16)) * 16
            zbuf[r, pl.ds(col, 16)] = jnp.zeros((16,), jnp.float32)
            return 0
        lax.fori_loop(0, K * (HALF // 16), fill, 0)

        def fill1(i, _):
            ones[pl.ds(i * 16, 16)] = jnp.ones((16,), jnp.float32)
            return 0
        lax.fori_loop(0, K // 16, fill1, 0)

        def fillz(i, _):
            zvec[pl.ds(i * 16, 16)] = jnp.zeros((16,), jnp.float32)
            return 0
        lax.fori_loop(0, 2 * ROWS_PER_TILE // 16, fillz, 0)

        def zero_sh(i, _):
            pltpu.sync_copy(zbuf, xagg_sh.at[pl.ds(sid * ROWS_PER_TILE + i * K, K)])
            return 0
        lax.fori_loop(0, ROWS_PER_TILE // K, zero_sh, 0)
        pltpu.sync_copy(
            zvec, hist_sh.at[pl.ds(sid * 2 * ROWS_PER_TILE, 2 * ROWS_PER_TILE)])
        plsc.subcore_barrier()

        pltpu.sync_copy(
            src_hbm.at[pl.ds(cid * EROWS + sid * BLOCKS_PER_TILE, BLOCKS_PER_TILE)],
            gidx)
        pltpu.sync_copy(dst_hbm.at[pl.ds(sid * BLOCKS_PER_TILE, BLOCKS_PER_TILE)],
                        sidx)

        rows = (rows0, rows1, rows2, rows3)
        sem_g = (sem_g0, sem_g1, sem_g2, sem_g3)
        sem_s = (sem_s0, sem_s1, sem_s2, sem_s3)
        def fire_hist(j):
            @pl.when(cid == 0)
            def _():
                pltpu.async_copy(ones, hist_sh.at[gidx.at[j]], sem_h, add=True)

        def wait_hist(j):
            @pl.when(cid == 0)
            def _():
                pltpu.make_async_copy(ones, hist_sh.at[gidx.at[j]], sem_h).wait()

        pltpu.async_copy(x_hbm.at[gidx.at[0]], rows0, sem_g0)
        pltpu.async_copy(x_hbm.at[gidx.at[1]], rows1, sem_g1)

        def body(i, _):
            for b in range(4):
                j = 4 * i + b
                bn = (b + 2) % 4
                pltpu.make_async_copy(x_hbm.at[gidx.at[j]], rows[b],
                                      sem_g[b]).wait()

                @pl.when(j >= 2)
                def _():
                    pltpu.make_async_copy(
                        rows[bn], xagg_sh.at[sidx.at[j - 2]], sem_s[bn]).wait()

                @pl.when(j + 2 < BLOCKS_PER_TILE)
                def _():
                    pltpu.async_copy(x_hbm.at[gidx.at[j + 2]], rows[bn],
                                     sem_g[bn])
                pltpu.async_copy(rows[b], xagg_sh.at[sidx.at[j]], sem_s[b],
                                 add=True)

                @pl.when(j > 0)
                def _():
                    wait_hist(j - 1)
                fire_hist(j)
            return 0
        lax.fori_loop(0, BLOCKS_PER_TILE // 4, body, 0)

        jl = BLOCKS_PER_TILE - 1
        pltpu.make_async_copy(rows[(jl - 1) % 4], xagg_sh.at[sidx.at[jl - 1]],
                              sem_s[(jl - 1) % 4]).wait()
        pltpu.make_async_copy(rows[jl % 4], xagg_sh.at[sidx.at[jl]],
                              sem_s[jl % 4]).wait()
        wait_hist(jl)
        plsc.subcore_barrier()

        base = cid * NPAD + sid * ROWS_PER_TILE
        pltpu.sync_copy(xagg_sh.at[pl.ds(sid * ROWS_PER_TILE, ROWS_PER_TILE)],
                        xagg_out.at[pl.ds(base, ROWS_PER_TILE)])

        @pl.when(cid == 0)
        def _():
            pltpu.sync_copy(
                hist_sh.at[pl.ds(sid * 2 * ROWS_PER_TILE, 2 * ROWS_PER_TILE)],
                c_out.at[pl.ds(sid * 2 * ROWS_PER_TILE, 2 * ROWS_PER_TILE)])

    return agg(x_cols, src_both, dst2d)


def _tc_h1_body(x_ref, w1_ref, b1_ref, h1_ref):
    h1_ref[...] = lax.dot_general(
        x_ref[...], w1_ref[...], (((1,), (1,)), ((), ())),
        preferred_element_type=jnp.float32) + b1_ref[...]


def _tc_finish_body(aglo_ref, aghi_ref, c_ref, w2_ref, b2_ref, out_ref):
    h2 = lax.dot_general(jnp.tanh(aglo_ref[...]), w2_ref[:, 0:HALF],
                         (((1,), (1,)), ((), ())),
                         preferred_element_type=jnp.float32)
    h2 = h2 + lax.dot_general(jnp.tanh(aghi_ref[...]), w2_ref[:, HALF:IN_DIM],
                              (((1,), (1,)), ((), ())),
                              preferred_element_type=jnp.float32)
    h2 = h2 + b2_ref[...]
    s = jnp.sum(h2, axis=1, keepdims=True)
    rowid = lax.broadcasted_iota(jnp.int32, (NPAD, 1), 0)
    c = jnp.where(rowid < N, c_ref[:, 0:1], 0.0)
    out_ref[...] = jnp.sum(c * s).reshape(1, 1)


def kernel(x, edge_index, W1, b1, W2, b2):
    x_pad = jnp.pad(x, ((0, NPAD - N), (0, 0)))

    h1 = pl.pallas_call(
        _tc_h1_body,
        out_shape=jax.ShapeDtypeStruct((NPAD, IN_DIM), jnp.float32),
    )(x_pad, W1, b1.reshape(1, HID_DIM))

    h1_cols = h1.reshape(2 * NPAD, HALF)
    pad = jnp.full((EPAD - E,), N, jnp.int32)
    src = jnp.concatenate([edge_index[0], pad])
    dst2d = jnp.concatenate([edge_index[1], pad]).reshape(EROWS, K)
    src_both = jnp.concatenate([2 * src, 2 * src + 1]).reshape(2 * EROWS, K)

    agg, c = _sc_aggregate(h1_cols, src_both, dst2d)

    out = pl.pallas_call(
        _tc_finish_body,
        out_shape=jax.ShapeDtypeStruct((1, 1), jnp.float32),
    )(agg[0:NPAD], agg[NPAD:2 * NPAD], c.reshape(NPAD, 2), W2,
      b2.reshape(1, OUT_DIM))
    return out


# speedup vs baseline: 1.0296x; 1.0296x over previous
import functools

import jax
import jax.numpy as jnp
from jax import lax
from jax.experimental import pallas as pl
from jax.experimental.pallas import tpu as pltpu
from jax.experimental.pallas import tpu_sc as plsc

N = 10000
E = 320000
IN_DIM = 128
HID_DIM = 128
OUT_DIM = 64
HALF = IN_DIM // 2

NPAD = 10240
EPAD = 327680
K = 128
BLOCKS_PER_TILE = 160
EROWS = EPAD // K
ROWS_PER_TILE = NPAD // 16
TC_BLK = 1280


def _sc_aggregate(x_cols, src_both, dst2d):
    mesh = plsc.VectorSubcoreMesh(core_axis_name="c", subcore_axis_name="s")

    @functools.partial(
        pl.kernel,
        mesh=mesh,
        compiler_params=pltpu.CompilerParams(use_tc_tiling_on_sc=False),
        out_type=[
            jax.ShapeDtypeStruct((2 * NPAD, HALF), jnp.float32),
            jax.ShapeDtypeStruct((2 * NPAD,), jnp.float32),
        ],
        scratch_types=[
            pltpu.VMEM((BLOCKS_PER_TILE, K), jnp.int32),
            pltpu.VMEM((BLOCKS_PER_TILE, K), jnp.int32),
        ] + [pltpu.VMEM((K, HALF), jnp.float32)] * 5 + [
            pltpu.VMEM((K,), jnp.float32),
            pltpu.VMEM((2 * ROWS_PER_TILE,), jnp.float32),
            pltpu.VMEM_SHARED((NPAD, HALF), jnp.float32),
            pltpu.VMEM_SHARED((2 * NPAD,), jnp.float32),
        ] + [pltpu.SemaphoreType.DMA] * 11,
    )
    def agg(x_hbm, src_hbm, dst_hbm, xagg_out, c_out,
            gidx, sidx, r0, r1, r2, r3, r4, ones, zvec,
            xagg_sh, hist_sh, g0, g1, g2, g3, g4,
            s0, s1, s2, s3, s4, sem_h):
        cid = lax.axis_index("c")
        sid = lax.axis_index("s")

        def fill(i, _):
            r = i // (HALF // 16)
            col = (i % (HALF // 16)) * 16
            r0[r, pl.ds(col, 16)] = jnp.zeros((16,), jnp.float32)
            return 0
        lax.fori_loop(0, K * (HALF // 16), fill, 0)

        def fill1(i, _):
            ones[pl.ds(i * 16, 16)] = jnp.ones((16,), jnp.float32)
            return 0
        lax.fori_loop(0, K // 16, fill1, 0)

        def fillz(i, _):
            zvec[pl.ds(i * 16, 16)] = jnp.zeros((16,), jnp.float32)
            return 0
        lax.fori_loop(0, 2 * ROWS_PER_TILE // 16, fillz, 0)

        def zero_sh(i, _):
            pltpu.sync_copy(r0, xagg_sh.at[pl.ds(sid * ROWS_PER_TILE + i * K, K)])
            return 0
        lax.fori_loop(0, ROWS_PER_TILE // K, zero_sh, 0)
        pltpu.sync_copy(
            zvec, hist_sh.at[pl.ds(sid * 2 * ROWS_PER_TILE, 2 * ROWS_PER_TILE)])
        plsc.subcore_barrier()

        pltpu.sync_copy(
            src_hbm.at[pl.ds(cid * EROWS + sid * BLOCKS_PER_TILE, BLOCKS_PER_TILE)],
            gidx)
        pltpu.sync_copy(dst_hbm.at[pl.ds(sid * BLOCKS_PER_TILE, BLOCKS_PER_TILE)],
                        sidx)

        rows = (r0, r1, r2, r3, r4)
        sem_g = (g0, g1, g2, g3, g4)
        sem_s = (s0, s1, s2, s3, s4)
        def fire_hist(j):
            @pl.when(cid == 0)
            def _():
                pltpu.async_copy(ones, hist_sh.at[gidx.at[j]], sem_h, add=True)

        def wait_hist(j):
            @pl.when(cid == 0)
            def _():
                pltpu.make_async_copy(ones, hist_sh.at[gidx.at[j]], sem_h).wait()

        for jj in range(3):
            pltpu.async_copy(x_hbm.at[gidx.at[jj]], rows[jj], sem_g[jj])

        def body(i, _):
            for b in range(5):
                j = 5 * i + b
                bn = (b + 3) % 5
                pltpu.make_async_copy(x_hbm.at[gidx.at[j]], rows[b],
                                      sem_g[b]).wait()

                @pl.when(j >= 2)
                def _():
                    pltpu.make_async_copy(
                        rows[bn], xagg_sh.at[sidx.at[j - 2]], sem_s[bn]).wait()

                @pl.when(j + 3 < BLOCKS_PER_TILE)
                def _():
                    pltpu.async_copy(x_hbm.at[gidx.at[j + 3]], rows[bn],
                                     sem_g[bn])
                pltpu.async_copy(rows[b], xagg_sh.at[sidx.at[j]], sem_s[b],
                                 add=True)

                @pl.when(j > 0)
                def _():
                    wait_hist(j - 1)
                fire_hist(j)
            return 0
        lax.fori_loop(0, BLOCKS_PER_TILE // 5, body, 0)

        jl = BLOCKS_PER_TILE - 1
        pltpu.make_async_copy(rows[(jl - 1) % 5], xagg_sh.at[sidx.at[jl - 1]],
                              sem_s[(jl - 1) % 5]).wait()
        pltpu.make_async_copy(rows[jl % 5], xagg_sh.at[sidx.at[jl]],
                              sem_s[jl % 5]).wait()
        wait_hist(jl)
        plsc.subcore_barrier()

        base = cid * NPAD + sid * ROWS_PER_TILE
        pltpu.sync_copy(xagg_sh.at[pl.ds(sid * ROWS_PER_TILE, ROWS_PER_TILE)],
                        xagg_out.at[pl.ds(base, ROWS_PER_TILE)])

        @pl.when(cid == 0)
        def _():
            pltpu.sync_copy(
                hist_sh.at[pl.ds(sid * 2 * ROWS_PER_TILE, 2 * ROWS_PER_TILE)],
                c_out.at[pl.ds(sid * 2 * ROWS_PER_TILE, 2 * ROWS_PER_TILE)])

    return agg(x_cols, src_both, dst2d)


def _tc_h1_body(x_ref, w1_ref, b1_ref, h1_ref):
    h1_ref[...] = lax.dot_general(
        x_ref[...], w1_ref[...], (((1,), (1,)), ((), ())),
        preferred_element_type=jnp.float32) + b1_ref[...]


def _tc_finish_body(aglo_ref, aghi_ref, c_ref, w2_ref, b2_ref, out_ref):
    h2 = lax.dot_general(jnp.tanh(aglo_ref[...]), w2_ref[:, 0:HALF],
                         (((1,), (1,)), ((), ())),
                         preferred_element_type=jnp.float32)
    h2 = h2 + lax.dot_general(jnp.tanh(aghi_ref[...]), w2_ref[:, HALF:IN_DIM],
                              (((1,), (1,)), ((), ())),
                              preferred_element_type=jnp.float32)
    h2 = h2 + b2_ref[...]
    s = jnp.sum(h2, axis=1, keepdims=True)
    rowid = lax.broadcasted_iota(jnp.int32, (NPAD, 1), 0)
    c = jnp.where(rowid < N, c_ref[:, 0:1], 0.0)
    out_ref[...] = jnp.sum(c * s).reshape(1, 1)


def kernel(x, edge_index, W1, b1, W2, b2):
    x_pad = jnp.pad(x, ((0, NPAD - N), (0, 0)))

    h1 = pl.pallas_call(
        _tc_h1_body,
        out_shape=jax.ShapeDtypeStruct((NPAD, IN_DIM), jnp.float32),
    )(x_pad, W1, b1.reshape(1, HID_DIM))

    h1_cols = h1.reshape(2 * NPAD, HALF)
    pad = jnp.full((EPAD - E,), N, jnp.int32)
    src = jnp.concatenate([edge_index[0], pad])
    dst2d = jnp.concatenate([edge_index[1], pad]).reshape(EROWS, K)
    src_both = jnp.concatenate([2 * src, 2 * src + 1]).reshape(2 * EROWS, K)

    agg, c = _sc_aggregate(h1_cols, src_both, dst2d)

    out = pl.pallas_call(
        _tc_finish_body,
        out_shape=jax.ShapeDtypeStruct((1, 1), jnp.float32),
    )(agg[0:NPAD], agg[NPAD:2 * NPAD], c.reshape(NPAD, 2), W2,
      b2.reshape(1, OUT_DIM))
    return out


# speedup vs baseline: 1.0399x; 1.0100x over previous
import functools

import jax
import jax.numpy as jnp
from jax import lax
from jax.experimental import pallas as pl
from jax.experimental.pallas import tpu as pltpu
from jax.experimental.pallas import tpu_sc as plsc

N = 10000
E = 320000
IN_DIM = 128
HID_DIM = 128
OUT_DIM = 64
HALF = IN_DIM // 2

NPAD = 10240
EPAD = 327680
K = 128
BLOCKS_PER_TILE = 160
EROWS = EPAD // K
ROWS_PER_TILE = NPAD // 16
TC_BLK = 1280


def _sc_aggregate(x_cols, src_both, dst2d):
    mesh = plsc.VectorSubcoreMesh(core_axis_name="c", subcore_axis_name="s")

    @functools.partial(
        pl.kernel,
        mesh=mesh,
        compiler_params=pltpu.CompilerParams(use_tc_tiling_on_sc=False),
        out_type=[
            jax.ShapeDtypeStruct((2 * NPAD, HALF), jnp.float32),
            jax.ShapeDtypeStruct((2 * NPAD,), jnp.float32),
        ],
        scratch_types=[
            pltpu.VMEM((BLOCKS_PER_TILE, K), jnp.int32),
            pltpu.VMEM((BLOCKS_PER_TILE, K), jnp.int32),
        ] + [pltpu.VMEM((K, HALF), jnp.float32)] * 5 + [
            pltpu.VMEM((K,), jnp.float32),
            pltpu.VMEM((2 * ROWS_PER_TILE,), jnp.float32),
            pltpu.VMEM_SHARED((NPAD, HALF), jnp.float32),
            pltpu.VMEM_SHARED((2 * NPAD,), jnp.float32),
        ] + [pltpu.SemaphoreType.DMA] * 11,
    )
    def agg(x_hbm, src_hbm, dst_hbm, xagg_out, c_out,
            gidx, sidx, r0, r1, r2, r3, r4, ones, zvec,
            xagg_sh, hist_sh, g0, g1, g2, g3, g4,
            s0, s1, s2, s3, s4, sem_h):
        cid = lax.axis_index("c")
        sid = lax.axis_index("s")

        def fill(i, _):
            r = i // (HALF // 16)
            col = (i % (HALF // 16)) * 16
            r0[r, pl.ds(col, 16)] = jnp.zeros((16,), jnp.float32)
            return 0
        lax.fori_loop(0, K * (HALF // 16), fill, 0)

        def fill1(i, _):
            ones[pl.ds(i * 16, 16)] = jnp.ones((16,), jnp.float32)
            return 0
        lax.fori_loop(0, K // 16, fill1, 0)

        def fillz(i, _):
            zvec[pl.ds(i * 16, 16)] = jnp.zeros((16,), jnp.float32)
            return 0
        lax.fori_loop(0, 2 * ROWS_PER_TILE // 16, fillz, 0)

        def zero_sh(i, _):
            pltpu.sync_copy(r0, xagg_sh.at[pl.ds(sid * ROWS_PER_TILE + i * K, K)])
            return 0
        lax.fori_loop(0, ROWS_PER_TILE // K, zero_sh, 0)
        pltpu.sync_copy(
            zvec, hist_sh.at[pl.ds(sid * 2 * ROWS_PER_TILE, 2 * ROWS_PER_TILE)])
        plsc.subcore_barrier()

        pltpu.sync_copy(
            src_hbm.at[pl.ds(cid * EROWS + sid * BLOCKS_PER_TILE, BLOCKS_PER_TILE)],
            gidx)
        pltpu.sync_copy(dst_hbm.at[pl.ds(sid * BLOCKS_PER_TILE, BLOCKS_PER_TILE)],
                        sidx)

        rows = (r0, r1, r2, r3, r4)
        sem_g = (g0, g1, g2, g3, g4)
        sem_s = (s0, s1, s2, s3, s4)
        def fire_hist(j):
            @pl.when(cid == 0)
            def _():
                pltpu.async_copy(ones, hist_sh.at[gidx.at[j]], sem_h, add=True)

        def wait_hist(j):
            @pl.when(cid == 0)
            def _():
                pltpu.make_async_copy(ones, hist_sh.at[gidx.at[j]], sem_h).wait()

        for jj in range(4):
            pltpu.async_copy(x_hbm.at[gidx.at[jj]], rows[jj], sem_g[jj])

        def body(i, _):
            for b in range(5):
                j = 5 * i + b
                bn = (b + 4) % 5
                pltpu.make_async_copy(x_hbm.at[gidx.at[j]], rows[b],
                                      sem_g[b]).wait()

                @pl.when(j >= 1)
                def _():
                    pltpu.make_async_copy(
                        rows[bn], xagg_sh.at[sidx.at[j - 1]], sem_s[bn]).wait()

                @pl.when(j + 4 < BLOCKS_PER_TILE)
                def _():
                    pltpu.async_copy(x_hbm.at[gidx.at[j + 4]], rows[bn],
                                     sem_g[bn])
                pltpu.async_copy(rows[b], xagg_sh.at[sidx.at[j]], sem_s[b],
                                 add=True)

                @pl.when(j > 0)
                def _():
                    wait_hist(j - 1)
                fire_hist(j)
            return 0
        lax.fori_loop(0, BLOCKS_PER_TILE // 5, body, 0)

        jl = BLOCKS_PER_TILE - 1
        pltpu.make_async_copy(rows[jl % 5], xagg_sh.at[sidx.at[jl]],
                              sem_s[jl % 5]).wait()
        wait_hist(jl)
        plsc.subcore_barrier()

        base = cid * NPAD + sid * ROWS_PER_TILE
        pltpu.sync_copy(xagg_sh.at[pl.ds(sid * ROWS_PER_TILE, ROWS_PER_TILE)],
                        xagg_out.at[pl.ds(base, ROWS_PER_TILE)])

        @pl.when(cid == 0)
        def _():
            pltpu.sync_copy(
                hist_sh.at[pl.ds(sid * 2 * ROWS_PER_TILE, 2 * ROWS_PER_TILE)],
                c_out.at[pl.ds(sid * 2 * ROWS_PER_TILE, 2 * ROWS_PER_TILE)])

    return agg(x_cols, src_both, dst2d)


def _tc_h1_body(x_ref, w1_ref, b1_ref, h1_ref):
    h1_ref[...] = lax.dot_general(
        x_ref[...], w1_ref[...], (((1,), (1,)), ((), ())),
        preferred_element_type=jnp.float32) + b1_ref[...]


def _tc_finish_body(aglo_ref, aghi_ref, c_ref, w2_ref, b2_ref, out_ref):
    h2 = lax.dot_general(jnp.tanh(aglo_ref[...]), w2_ref[:, 0:HALF],
                         (((1,), (1,)), ((), ())),
                         preferred_element_type=jnp.float32)
    h2 = h2 + lax.dot_general(jnp.tanh(aghi_ref[...]), w2_ref[:, HALF:IN_DIM],
                              (((1,), (1,)), ((), ())),
                              preferred_element_type=jnp.float32)
    h2 = h2 + b2_ref[...]
    s = jnp.sum(h2, axis=1, keepdims=True)
    rowid = lax.broadcasted_iota(jnp.int32, (NPAD, 1), 0)
    c = jnp.where(rowid < N, c_ref[:, 0:1], 0.0)
    out_ref[...] = jnp.sum(c * s).reshape(1, 1)


def kernel(x, edge_index, W1, b1, W2, b2):
    x_pad = jnp.pad(x, ((0, NPAD - N), (0, 0)))

    h1 = pl.pallas_call(
        _tc_h1_body,
        out_shape=jax.ShapeDtypeStruct((NPAD, IN_DIM), jnp.float32),
    )(x_pad, W1, b1.reshape(1, HID_DIM))

    h1_cols = h1.reshape(2 * NPAD, HALF)
    pad = jnp.full((EPAD - E,), N, jnp.int32)
    src = jnp.concatenate([edge_index[0], pad])
    dst2d = jnp.concatenate([edge_index[1], pad]).reshape(EROWS, K)
    src_both = jnp.concatenate([2 * src, 2 * src + 1]).reshape(2 * EROWS, K)

    agg, c = _sc_aggregate(h1_cols, src_both, dst2d)

    out = pl.pallas_call(
        _tc_finish_body,
        out_shape=jax.ShapeDtypeStruct((1, 1), jnp.float32),
    )(agg[0:NPAD], agg[NPAD:2 * NPAD], c.reshape(NPAD, 2), W2,
      b2.reshape(1, OUT_DIM))
    return out
